# Initial kernel scaffold; baseline (speedup 1.0000x reference)
import jax
import jax.numpy as jnp
from jax import lax
from jax.experimental import pallas as pl
from jax.experimental.pallas import tpu as pltpu
from jax.experimental.pallas import tpu_sc as plsc

N = 50000
E = 800000
DIM = 32
NC = 2
NS = 16
CHUNK = 128
CPT = 392
EPT = CPT * CHUNK
EPAD = NS * EPT
RPT = 3200
NPAD = NS * RPT
NB = 1024
GB = NPAD // NB


def _embed_body(vals_ref, tab_ref, out_ref):
    v = vals_ref[0, 0]
    iot = lax.broadcasted_iota(jnp.int32, (64, NB), 0)
    oh = (iot == v).astype(jnp.float32)
    out_ref[0] = lax.dot_general(
        oh, tab_ref[0], (((0,), (0,)), ((), ())),
        preferred_element_type=jnp.float32)


def _embed(vals, tabs):
    return pl.pallas_call(
        _embed_body,
        grid=(2, GB),
        in_specs=[
            pl.BlockSpec((1, 1, 1, NB), lambda i, j: (i, j, 0, 0)),
            pl.BlockSpec((1, 64, DIM), lambda i, j: (i, 0, 0)),
        ],
        out_specs=pl.BlockSpec((1, NB, DIM), lambda i, j: (i, j, 0)),
        out_shape=jax.ShapeDtypeStruct((2, NPAD, DIM), jnp.float32),
    )(vals, tabs)


def _sc_segsum_body(src_hbm, dst_hbm, tmp_hbm, agg_hbm,
                    src_v, dst_v, rows_a, rows_b, agg_sh,
                    sem_ga, sem_gb, sem_sa, sem_sb):
    c = lax.axis_index("c")
    s = lax.axis_index("s")
    z16 = jnp.zeros((16,), jnp.float32)

    def zrow(i, carry):
        rows_a[i, pl.ds(0, 16)] = z16
        rows_a[i, pl.ds(16, 16)] = z16
        return carry

    lax.fori_loop(0, CHUNK, zrow, 0)

    def zblk(g, carry):
        pltpu.sync_copy(rows_a,
                        agg_sh.at[pl.ds((s * (RPT // CHUNK) + g) * CHUNK, CHUNK)])
        return carry

    lax.fori_loop(0, RPT // CHUNK, zblk, 0)

    pltpu.sync_copy(src_hbm.at[c, s], src_v)
    pltpu.sync_copy(dst_hbm.at[s], dst_v)
    plsc.subcore_barrier()

    pltpu.async_copy(tmp_hbm.at[src_v.at[0]], rows_a, sem_ga)
    pltpu.async_copy(tmp_hbm.at[src_v.at[1]], rows_b, sem_gb)

    def body(i, carry):
        g0 = 2 * i
        g1 = 2 * i + 1
        pltpu.make_async_copy(tmp_hbm.at[src_v.at[g0]], rows_a, sem_ga).wait()
        pltpu.async_copy(rows_a, agg_sh.at[dst_v.at[g0]], sem_sa, add=True)
        pltpu.make_async_copy(tmp_hbm.at[src_v.at[g1]], rows_b, sem_gb).wait()
        pltpu.async_copy(rows_b, agg_sh.at[dst_v.at[g1]], sem_sb, add=True)
        pltpu.make_async_copy(rows_a, agg_sh.at[dst_v.at[g0]], sem_sa).wait()

        @pl.when(i < CPT // 2 - 1)
        def _():
            pltpu.async_copy(tmp_hbm.at[src_v.at[g0 + 2]], rows_a, sem_ga)

        pltpu.make_async_copy(rows_b, agg_sh.at[dst_v.at[g1]], sem_sb).wait()

        @pl.when(i < CPT // 2 - 1)
        def _():
            pltpu.async_copy(tmp_hbm.at[src_v.at[g1 + 2]], rows_b, sem_gb)

        return carry

    lax.fori_loop(0, CPT // 2, body, 0)
    plsc.subcore_barrier()
    pltpu.sync_copy(agg_sh.at[pl.ds(s * RPT, RPT)],
                    agg_hbm.at[c, pl.ds(s * RPT, RPT)])


def _sc_segsum(src_off, dst2d, tmp_flat):
    mesh = plsc.VectorSubcoreMesh(core_axis_name="c", subcore_axis_name="s",
                                  num_cores=NC, num_subcores=NS)
    return pl.kernel(
        _sc_segsum_body,
        out_type=jax.ShapeDtypeStruct((NC, NPAD, DIM), jnp.float32),
        mesh=mesh,
        scratch_types=[
            pltpu.VMEM((CPT, CHUNK), jnp.int32),
            pltpu.VMEM((CPT, CHUNK), jnp.int32),
            pltpu.VMEM((CHUNK, DIM), jnp.float32),
            pltpu.VMEM((CHUNK, DIM), jnp.float32),
            pltpu.VMEM_SHARED((NPAD, DIM), jnp.float32),
            pltpu.SemaphoreType.DMA,
            pltpu.SemaphoreType.DMA,
            pltpu.SemaphoreType.DMA,
            pltpu.SemaphoreType.DMA,
        ],
    )(src_off, dst2d, tmp_flat)


def _xlayer_body(td_ref, tl_ref, ad_ref, al_ref, w_ref, b_ref, x_ref, st_ref):
    j = pl.program_id(0)
    h = jnp.concatenate([td_ref[0] + ad_ref[0], tl_ref[0] + al_ref[0]], axis=1)
    x = jnp.dot(h, w_ref[...], preferred_element_type=jnp.float32) + b_ref[...]
    x_ref[...] = x
    rows = j * NB + lax.broadcasted_iota(jnp.int32, (NB, 1), 0)
    xm = jnp.where(rows < N, x, 0.0)

    @pl.when(j == 0)
    def _():
        st_ref[...] = jnp.zeros_like(st_ref)

    st_ref[0:1, :] += jnp.sum(xm, axis=0, keepdims=True)
    st_ref[1:2, :] += jnp.sum(xm * xm, axis=0, keepdims=True)


def _xlayer(tmp_both, agg_both, W0, b0):
    return pl.pallas_call(
        _xlayer_body,
        grid=(GB,),
        in_specs=[
            pl.BlockSpec((1, NB, DIM), lambda j: (0, j, 0)),
            pl.BlockSpec((1, NB, DIM), lambda j: (1, j, 0)),
            pl.BlockSpec((1, NB, DIM), lambda j: (0, j, 0)),
            pl.BlockSpec((1, NB, DIM), lambda j: (1, j, 0)),
            pl.BlockSpec((2 * DIM, DIM), lambda j: (0, 0)),
            pl.BlockSpec((1, DIM), lambda j: (0, 0)),
        ],
        out_specs=[
            pl.BlockSpec((NB, DIM), lambda j: (j, 0)),
            pl.BlockSpec((8, DIM), lambda j: (0, 0)),
        ],
        out_shape=[
            jax.ShapeDtypeStruct((NPAD, DIM), jnp.float32),
            jax.ShapeDtypeStruct((8, DIM), jnp.float32),
        ],
    )(tmp_both, tmp_both, agg_both, agg_both, W0, b0)


def _leaky(x):
    return jnp.where(x >= 0, x, 0.01 * x)


def _ylayer_body(st_ref, x_ref, td_ref, tl_ref, g_ref, b_ref, w_ref, b1_ref,
                 y_ref, sty_ref):
    j = pl.program_id(0)
    m = st_ref[0:1, :] * (1.0 / N)
    var = st_ref[1:2, :] * (1.0 / N) - m * m
    inv = lax.rsqrt(var + 1e-5)
    xn = _leaky((x_ref[...] - m) * inv * g_ref[...] + b_ref[...])
    xc = jnp.concatenate([td_ref[0], tl_ref[0], xn], axis=1)
    y = jnp.dot(xc, w_ref[...], preferred_element_type=jnp.float32) + b1_ref[...]
    y_ref[...] = y
    rows = j * NB + lax.broadcasted_iota(jnp.int32, (NB, 1), 0)
    ym = jnp.where(rows < N, y, 0.0)

    @pl.when(j == 0)
    def _():
        sty_ref[...] = jnp.zeros_like(sty_ref)

    sty_ref[0:1, :] += jnp.sum(ym, axis=0, keepdims=True)
    sty_ref[1:2, :] += jnp.sum(ym * ym, axis=0, keepdims=True)


def _ylayer(st_x, x, tmp_both, bn0_g, bn0_b, fc_W1, fc_b1):
    return pl.pallas_call(
        _ylayer_body,
        grid=(GB,),
        in_specs=[
            pl.BlockSpec((8, DIM), lambda j: (0, 0)),
            pl.BlockSpec((NB, DIM), lambda j: (j, 0)),
            pl.BlockSpec((1, NB, DIM), lambda j: (0, j, 0)),
            pl.BlockSpec((1, NB, DIM), lambda j: (1, j, 0)),
            pl.BlockSpec((1, DIM), lambda j: (0, 0)),
            pl.BlockSpec((1, DIM), lambda j: (0, 0)),
            pl.BlockSpec((3 * DIM, DIM), lambda j: (0, 0)),
            pl.BlockSpec((1, DIM), lambda j: (0, 0)),
        ],
        out_specs=[
            pl.BlockSpec((NB, DIM), lambda j: (j, 0)),
            pl.BlockSpec((8, DIM), lambda j: (0, 0)),
        ],
        out_shape=[
            jax.ShapeDtypeStruct((NPAD, DIM), jnp.float32),
            jax.ShapeDtypeStruct((8, DIM), jnp.float32),
        ],
    )(st_x, x, tmp_both, tmp_both, bn0_g, bn0_b, fc_W1, fc_b1)


def _zlayer_body(st_ref, y_ref, g_ref, b_ref, w2_ref, b2_ref, z_ref):
    m = st_ref[0:1, :] * (1.0 / N)
    var = st_ref[1:2, :] * (1.0 / N) - m * m
    inv = lax.rsqrt(var + 1e-5)
    yn = _leaky((y_ref[...] - m) * inv * g_ref[...] + b_ref[...])
    z = jnp.sum(yn * w2_ref[...], axis=1, keepdims=True) + b2_ref[...]
    z_ref[...] = 1.0 / (1.0 + jnp.exp(-z))


def _zlayer(st_y, y, fc_bn_g, fc_bn_b, w2r, b2r):
    return pl.pallas_call(
        _zlayer_body,
        grid=(GB,),
        in_specs=[
            pl.BlockSpec((8, DIM), lambda j: (0, 0)),
            pl.BlockSpec((NB, DIM), lambda j: (j, 0)),
            pl.BlockSpec((1, DIM), lambda j: (0, 0)),
            pl.BlockSpec((1, DIM), lambda j: (0, 0)),
            pl.BlockSpec((1, DIM), lambda j: (0, 0)),
            pl.BlockSpec((1, 1), lambda j: (0, 0)),
        ],
        out_specs=pl.BlockSpec((NB, 1), lambda j: (j, 0)),
        out_shape=jax.ShapeDtypeStruct((NPAD, 1), jnp.float32),
    )(st_y, y, fc_bn_g, fc_bn_b, w2r, b2r)


def kernel(node_deg, node_lab, edge_index, embed_deg_w, embed_lab_w,
           W0, b0, bn0_g, bn0_b, fc_W1, fc_b1, fc_bn_g, fc_bn_b, fc_W2, fc_b2):
    i32 = jnp.int32
    npad = NPAD - N
    vals = jnp.stack([
        jnp.concatenate([node_deg.astype(i32), jnp.zeros((npad,), i32)]),
        jnp.concatenate([node_lab.astype(i32), jnp.zeros((npad,), i32)]),
    ]).reshape(2, GB, 1, NB)
    tabs = jnp.stack([embed_deg_w,
                      jnp.pad(embed_lab_w, ((0, 64 - embed_lab_w.shape[0]), (0, 0)))])
    tmp_both = _embed(vals, tabs)

    epad = EPAD - E
    srcp = jnp.concatenate([edge_index[0].astype(i32), jnp.zeros((epad,), i32)])
    dstp = jnp.concatenate([edge_index[1].astype(i32), jnp.full((epad,), N, i32)])
    src_off = jnp.stack([srcp, srcp + NPAD]).reshape(NC, NS, CPT, CHUNK)
    dst2d = dstp.reshape(NS, CPT, CHUNK)
    tmp_flat = tmp_both.reshape(NC * NPAD, DIM)

    agg_both = _sc_segsum(src_off, dst2d, tmp_flat)

    x, st_x = _xlayer(tmp_both, agg_both, W0, b0.reshape(1, DIM))
    y, st_y = _ylayer(st_x, x, tmp_both, bn0_g.reshape(1, DIM),
                      bn0_b.reshape(1, DIM), fc_W1, fc_b1.reshape(1, DIM))
    z = _zlayer(st_y, y, fc_bn_g.reshape(1, DIM), fc_bn_b.reshape(1, DIM),
                fc_W2.reshape(1, DIM), fc_b2.reshape(1, 1))
    return z[:N, 0]


# speedup vs baseline: 2.9108x; 2.9108x over previous
import jax
import jax.numpy as jnp
from jax import lax
from jax.experimental import pallas as pl
from jax.experimental.pallas import tpu as pltpu
from jax.experimental.pallas import tpu_sc as plsc

N = 50000
E = 800000
DIM = 32
QD = 8
NQ = 8
PPC = 4
NC = 2
NS = 16
CHUNK = 128
CPT = 392
EPT = CPT * CHUNK
EPAD = NS * EPT
RPT = 3200
NPAD = NS * RPT
NB = 1024
GB = NPAD // NB


def _embed_body(vals_ref, tab_ref, out_ref):
    v = vals_ref[0, 0]
    iot = lax.broadcasted_iota(jnp.int32, (64, NB), 0)
    oh = (iot == v).astype(jnp.float32)
    out_ref[0] = lax.dot_general(
        oh, tab_ref[0], (((0,), (0,)), ((), ())),
        preferred_element_type=jnp.float32)


def _embed(vals, tabs8):
    return pl.pallas_call(
        _embed_body,
        grid=(NQ, GB),
        in_specs=[
            pl.BlockSpec((1, 1, 1, NB), lambda q, j: (q // PPC, j, 0, 0)),
            pl.BlockSpec((1, 64, QD), lambda q, j: (q, 0, 0)),
        ],
        out_specs=pl.BlockSpec((1, NB, QD), lambda q, j: (q, j, 0)),
        out_shape=jax.ShapeDtypeStruct((NQ, NPAD, QD), jnp.float32),
    )(vals, tabs8)


def _sc_segsum_body(src_hbm, dst_hbm, tmp_hbm, zer_hbm, agg_hbm,
                    src_v, dst_v, rows_a, rows_b, agg_sh,
                    sem_ga, sem_gb, sem_sa, sem_sb):
    c = lax.axis_index("c")
    s = lax.axis_index("s")

    pltpu.sync_copy(src_hbm.at[s], src_v)
    pltpu.sync_copy(dst_hbm.at[s], dst_v)

    for j in range(PPC):
        q = PPC * c + j
        pltpu.sync_copy(zer_hbm, agg_sh.at[pl.ds(s * RPT, RPT)])
        plsc.subcore_barrier()

        tmp_q = tmp_hbm.at[q]
        pltpu.async_copy(tmp_q.at[src_v.at[0]], rows_a, sem_ga)
        pltpu.async_copy(tmp_q.at[src_v.at[1]], rows_b, sem_gb)

        def body(i, carry):
            g0 = 2 * i
            g1 = 2 * i + 1
            pltpu.make_async_copy(tmp_q.at[src_v.at[g0]], rows_a, sem_ga).wait()
            pltpu.async_copy(rows_a, agg_sh.at[dst_v.at[g0]], sem_sa, add=True)
            pltpu.make_async_copy(tmp_q.at[src_v.at[g1]], rows_b, sem_gb).wait()
            pltpu.async_copy(rows_b, agg_sh.at[dst_v.at[g1]], sem_sb, add=True)
            pltpu.make_async_copy(rows_a, agg_sh.at[dst_v.at[g0]], sem_sa).wait()

            @pl.when(i < CPT // 2 - 1)
            def _():
                pltpu.async_copy(tmp_q.at[src_v.at[g0 + 2]], rows_a, sem_ga)

            pltpu.make_async_copy(rows_b, agg_sh.at[dst_v.at[g1]], sem_sb).wait()

            @pl.when(i < CPT // 2 - 1)
            def _():
                pltpu.async_copy(tmp_q.at[src_v.at[g1 + 2]], rows_b, sem_gb)

            return carry

        lax.fori_loop(0, CPT // 2, body, 0)
        plsc.subcore_barrier()
        pltpu.sync_copy(agg_sh.at[pl.ds(s * RPT, RPT)],
                        agg_hbm.at[q, pl.ds(s * RPT, RPT)])


def _sc_segsum(src2d, dst2d, tmp8, zer):
    mesh = plsc.VectorSubcoreMesh(core_axis_name="c", subcore_axis_name="s",
                                  num_cores=NC, num_subcores=NS)
    return pl.kernel(
        _sc_segsum_body,
        out_type=pltpu.HBM((NQ, NPAD, QD), jnp.float32),
        mesh=mesh,
        compiler_params=pltpu.CompilerParams(use_tc_tiling_on_sc=False),
        scratch_types=[
            pltpu.VMEM((CPT, CHUNK), jnp.int32),
            pltpu.VMEM((CPT, CHUNK), jnp.int32),
            pltpu.VMEM((CHUNK, QD), jnp.float32),
            pltpu.VMEM((CHUNK, QD), jnp.float32),
            pltpu.VMEM_SHARED((NPAD, QD), jnp.float32),
            pltpu.SemaphoreType.DMA,
            pltpu.SemaphoreType.DMA,
            pltpu.SemaphoreType.DMA,
            pltpu.SemaphoreType.DMA,
        ],
    )(src2d, dst2d, tmp8, zer)


def _xlayer_body(*refs):
    ts = refs[0:NQ]
    As = refs[NQ:2 * NQ]
    w_ref, b_ref, x_ref, st_ref = refs[2 * NQ:]
    j = pl.program_id(0)
    h = jnp.concatenate([ts[q][0] + As[q][0] for q in range(NQ)], axis=1)
    x = jnp.dot(h, w_ref[...], preferred_element_type=jnp.float32) + b_ref[...]
    x_ref[...] = x
    rows = j * NB + lax.broadcasted_iota(jnp.int32, (NB, 1), 0)
    xm = jnp.where(rows < N, x, 0.0)

    @pl.when(j == 0)
    def _():
        st_ref[...] = jnp.zeros_like(st_ref)

    st_ref[0:1, :] += jnp.sum(xm, axis=0, keepdims=True)
    st_ref[1:2, :] += jnp.sum(xm * xm, axis=0, keepdims=True)


def _qspec(q):
    return pl.BlockSpec((1, NB, QD), lambda j, q=q: (q, j, 0))


def _xlayer(tmp8, agg8, W0, b0):
    return pl.pallas_call(
        _xlayer_body,
        grid=(GB,),
        in_specs=([_qspec(q) for q in range(NQ)]
                  + [_qspec(q) for q in range(NQ)]
                  + [pl.BlockSpec((2 * DIM, DIM), lambda j: (0, 0)),
                     pl.BlockSpec((1, DIM), lambda j: (0, 0))]),
        out_specs=[
            pl.BlockSpec((NB, DIM), lambda j: (j, 0)),
            pl.BlockSpec((8, DIM), lambda j: (0, 0)),
        ],
        out_shape=[
            jax.ShapeDtypeStruct((NPAD, DIM), jnp.float32),
            jax.ShapeDtypeStruct((8, DIM), jnp.float32),
        ],
    )(*([tmp8] * NQ), *([agg8] * NQ), W0, b0)


def _leaky(x):
    return jnp.where(x >= 0, x, 0.01 * x)


def _ylayer_body(*refs):
    st_ref, x_ref = refs[0], refs[1]
    ts = refs[2:2 + NQ]
    g_ref, b_ref, w_ref, b1_ref, y_ref, sty_ref = refs[2 + NQ:]
    j = pl.program_id(0)
    m = st_ref[0:1, :] * (1.0 / N)
    var = st_ref[1:2, :] * (1.0 / N) - m * m
    inv = lax.rsqrt(var + 1e-5)
    xn = _leaky((x_ref[...] - m) * inv * g_ref[...] + b_ref[...])
    xc = jnp.concatenate([ts[q][0] for q in range(NQ)] + [xn], axis=1)
    y = jnp.dot(xc, w_ref[...], preferred_element_type=jnp.float32) + b1_ref[...]
    y_ref[...] = y
    rows = j * NB + lax.broadcasted_iota(jnp.int32, (NB, 1), 0)
    ym = jnp.where(rows < N, y, 0.0)

    @pl.when(j == 0)
    def _():
        sty_ref[...] = jnp.zeros_like(sty_ref)

    sty_ref[0:1, :] += jnp.sum(ym, axis=0, keepdims=True)
    sty_ref[1:2, :] += jnp.sum(ym * ym, axis=0, keepdims=True)


def _ylayer(st_x, x, tmp8, bn0_g, bn0_b, fc_W1, fc_b1):
    return pl.pallas_call(
        _ylayer_body,
        grid=(GB,),
        in_specs=([pl.BlockSpec((8, DIM), lambda j: (0, 0)),
                   pl.BlockSpec((NB, DIM), lambda j: (j, 0))]
                  + [_qspec(q) for q in range(NQ)]
                  + [pl.BlockSpec((1, DIM), lambda j: (0, 0)),
                     pl.BlockSpec((1, DIM), lambda j: (0, 0)),
                     pl.BlockSpec((3 * DIM, DIM), lambda j: (0, 0)),
                     pl.BlockSpec((1, DIM), lambda j: (0, 0))]),
        out_specs=[
            pl.BlockSpec((NB, DIM), lambda j: (j, 0)),
            pl.BlockSpec((8, DIM), lambda j: (0, 0)),
        ],
        out_shape=[
            jax.ShapeDtypeStruct((NPAD, DIM), jnp.float32),
            jax.ShapeDtypeStruct((8, DIM), jnp.float32),
        ],
    )(st_x, x, *([tmp8] * NQ), bn0_g, bn0_b, fc_W1, fc_b1)


def _zlayer_body(st_ref, y_ref, g_ref, b_ref, w2_ref, b2_ref, z_ref):
    m = st_ref[0:1, :] * (1.0 / N)
    var = st_ref[1:2, :] * (1.0 / N) - m * m
    inv = lax.rsqrt(var + 1e-5)
    yn = _leaky((y_ref[...] - m) * inv * g_ref[...] + b_ref[...])
    z = jnp.sum(yn * w2_ref[...], axis=1, keepdims=True) + b2_ref[...]
    z_ref[...] = 1.0 / (1.0 + jnp.exp(-z))


def _zlayer(st_y, y, fc_bn_g, fc_bn_b, w2r, b2r):
    return pl.pallas_call(
        _zlayer_body,
        grid=(GB,),
        in_specs=[
            pl.BlockSpec((8, DIM), lambda j: (0, 0)),
            pl.BlockSpec((NB, DIM), lambda j: (j, 0)),
            pl.BlockSpec((1, DIM), lambda j: (0, 0)),
            pl.BlockSpec((1, DIM), lambda j: (0, 0)),
            pl.BlockSpec((1, DIM), lambda j: (0, 0)),
            pl.BlockSpec((1, 1), lambda j: (0, 0)),
        ],
        out_specs=pl.BlockSpec((NB, 1), lambda j: (j, 0)),
        out_shape=jax.ShapeDtypeStruct((NPAD, 1), jnp.float32),
    )(st_y, y, fc_bn_g, fc_bn_b, w2r, b2r)


def kernel(node_deg, node_lab, edge_index, embed_deg_w, embed_lab_w,
           W0, b0, bn0_g, bn0_b, fc_W1, fc_b1, fc_bn_g, fc_bn_b, fc_W2, fc_b2):
    i32 = jnp.int32
    npad = NPAD - N
    vals = jnp.stack([
        jnp.concatenate([node_deg.astype(i32), jnp.zeros((npad,), i32)]),
        jnp.concatenate([node_lab.astype(i32), jnp.zeros((npad,), i32)]),
    ]).reshape(2, GB, 1, NB)
    tabs = jnp.stack([embed_deg_w,
                      jnp.pad(embed_lab_w, ((0, 64 - embed_lab_w.shape[0]), (0, 0)))])
    tabs8 = tabs.reshape(2, 64, PPC, QD).transpose(0, 2, 1, 3).reshape(NQ, 64, QD)
    tmp8 = _embed(vals, tabs8)

    epad = EPAD - E
    srcp = jnp.concatenate([edge_index[0].astype(i32), jnp.zeros((epad,), i32)])
    dstp = jnp.concatenate([edge_index[1].astype(i32), jnp.full((epad,), N, i32)])
    src2d = srcp.reshape(NS, CPT, CHUNK)
    dst2d = dstp.reshape(NS, CPT, CHUNK)
    zer = jnp.zeros((RPT, QD), jnp.float32)

    agg8 = _sc_segsum(src2d, dst2d, tmp8, zer)

    x, st_x = _xlayer(tmp8, agg8, W0, b0.reshape(1, DIM))
    y, st_y = _ylayer(st_x, x, tmp8, bn0_g.reshape(1, DIM),
                      bn0_b.reshape(1, DIM), fc_W1, fc_b1.reshape(1, DIM))
    z = _zlayer(st_y, y, fc_bn_g.reshape(1, DIM), fc_bn_b.reshape(1, DIM),
                fc_W2.reshape(1, DIM), fc_b2.reshape(1, 1))
    return z[:N, 0]


# speedup vs baseline: 3.0707x; 1.0550x over previous
import jax
import jax.numpy as jnp
from jax import lax
from jax.experimental import pallas as pl
from jax.experimental.pallas import tpu as pltpu
from jax.experimental.pallas import tpu_sc as plsc

N = 50000
E = 800000
DIM = 32
QD = 8
NQ = 8
PPC = 4
NC = 2
NS = 16
CHUNK = 128
GRP = 2
CPT = 392
NG = CPT // GRP
EPT = CPT * CHUNK
EPAD = NS * EPT
RPT = 3200
NPAD = NS * RPT
EMB_H = 1600
IL = 16
NR = NPAD // IL
RB = 64
GB = NR // RB
NVAL = N // IL


def _sc_embed_body(vals_hbm, tabs_hbm, tmp_hbm, val_v, emb_v, sem):
    c = lax.axis_index("c")
    s = lax.axis_index("s")
    pltpu.sync_copy(vals_hbm.at[c, s], val_v)
    for j in range(PPC):
        q = PPC * c + j
        for h in range(RPT // EMB_H):
            off = val_v.at[pl.ds(h * EMB_H, EMB_H)]
            pltpu.async_copy(tabs_hbm.at[q].at[off], emb_v, sem)
            pltpu.make_async_copy(tabs_hbm.at[q].at[off], emb_v, sem).wait()
            pltpu.sync_copy(emb_v,
                            tmp_hbm.at[q, pl.ds(s * RPT + h * EMB_H, EMB_H)])


def _sc_embed(vals2, tabs8):
    mesh = plsc.VectorSubcoreMesh(core_axis_name="c", subcore_axis_name="s",
                                  num_cores=NC, num_subcores=NS)
    return pl.kernel(
        _sc_embed_body,
        out_type=pltpu.HBM((NQ, NPAD, QD), jnp.float32),
        mesh=mesh,
        compiler_params=pltpu.CompilerParams(use_tc_tiling_on_sc=False),
        scratch_types=[
            pltpu.VMEM((RPT,), jnp.int32),
            pltpu.VMEM((EMB_H, QD), jnp.float32),
            pltpu.SemaphoreType.DMA,
        ],
    )(vals2, tabs8)


def _sc_body(src_hbm, dst_hbm, tmp_hbm, zer_hbm, agg_hbm,
             src_v, dst_v, rows_a, rows_b, agg_sh,
             sem_ga, sem_gb, sem_sa, sem_sb):
    c = lax.axis_index("c")
    s = lax.axis_index("s")
    GC = GRP * CHUNK

    pltpu.sync_copy(src_hbm.at[s], src_v)
    pltpu.sync_copy(dst_hbm.at[s], dst_v)

    for j in range(PPC):
        q = PPC * c + j
        pltpu.sync_copy(zer_hbm, agg_sh.at[pl.ds(s * RPT, RPT)])
        plsc.subcore_barrier()

        tmp_q = tmp_hbm.at[q]
        pltpu.async_copy(tmp_q.at[src_v.at[pl.ds(0, GC)]], rows_a, sem_ga)
        pltpu.async_copy(tmp_q.at[src_v.at[pl.ds(GC, GC)]], rows_b, sem_gb)

        def body(i, carry):
            g0 = 2 * i
            g1 = 2 * i + 1
            sa = src_v.at[pl.ds(g0 * GC, GC)]
            sb = src_v.at[pl.ds(g1 * GC, GC)]

            def scat(rows, g, sem):
                def sk(k, carry):
                    pltpu.async_copy(rows.at[pl.ds(k * CHUNK, CHUNK)],
                                     agg_sh.at[dst_v.at[g * GRP + k]],
                                     sem, add=True)
                    return carry
                lax.fori_loop(0, GRP, sk, 0)

            def scat_wait(rows, g, sem):
                def sk(k, carry):
                    pltpu.make_async_copy(rows.at[pl.ds(k * CHUNK, CHUNK)],
                                          agg_sh.at[dst_v.at[g * GRP + k]],
                                          sem).wait()
                    return carry
                lax.fori_loop(0, GRP, sk, 0)

            pltpu.make_async_copy(tmp_q.at[sa], rows_a, sem_ga).wait()
            scat(rows_a, g0, sem_sa)
            pltpu.make_async_copy(tmp_q.at[sb], rows_b, sem_gb).wait()
            scat(rows_b, g1, sem_sb)
            scat_wait(rows_a, g0, sem_sa)

            @pl.when(i < NG // 2 - 1)
            def _():
                pltpu.async_copy(tmp_q.at[src_v.at[pl.ds((g0 + 2) * GC, GC)]],
                                 rows_a, sem_ga)

            scat_wait(rows_b, g1, sem_sb)

            @pl.when(i < NG // 2 - 1)
            def _():
                pltpu.async_copy(tmp_q.at[src_v.at[pl.ds((g1 + 2) * GC, GC)]],
                                 rows_b, sem_gb)

            return carry

        lax.fori_loop(0, NG // 2, body, 0)
        plsc.subcore_barrier()
        pltpu.sync_copy(agg_sh.at[pl.ds(s * RPT, RPT)],
                        agg_hbm.at[q, pl.ds(s * RPT, RPT)])


def _sc_segsum(src2d, dst2d, tmp8, zer):
    mesh = plsc.VectorSubcoreMesh(core_axis_name="c", subcore_axis_name="s",
                                  num_cores=NC, num_subcores=NS)
    return pl.kernel(
        _sc_body,
        out_type=pltpu.HBM((NQ, NPAD, QD), jnp.float32),
        mesh=mesh,
        compiler_params=pltpu.CompilerParams(use_tc_tiling_on_sc=False),
        scratch_types=[
            pltpu.VMEM((EPT,), jnp.int32),
            pltpu.VMEM((CPT, CHUNK), jnp.int32),
            pltpu.VMEM((GRP * CHUNK, QD), jnp.float32),
            pltpu.VMEM((GRP * CHUNK, QD), jnp.float32),
            pltpu.VMEM_SHARED((NPAD, QD), jnp.float32),
            pltpu.SemaphoreType.DMA,
            pltpu.SemaphoreType.DMA,
            pltpu.SemaphoreType.DMA,
            pltpu.SemaphoreType.DMA,
        ],
    )(src2d, dst2d, tmp8, zer)


def _leaky(x):
    return jnp.where(x >= 0, x, 0.01 * x)


def _iqspec(q):
    return pl.BlockSpec((1, RB, 128), lambda j, q=q: (q, j, 0))


def _cspec(shape):
    return pl.BlockSpec(shape, lambda j, shape=shape: tuple(0 for _ in shape))


def _xlayer_body(*refs):
    ts = refs[0:NQ]
    As = refs[NQ:2 * NQ]
    w_ref, b_ref, x_ref, st_ref = refs[2 * NQ:]
    j = pl.program_id(0)
    h = jnp.concatenate([ts[q][0] + As[q][0] for q in range(NQ)], axis=1)
    x = jnp.dot(h, w_ref[...], preferred_element_type=jnp.float32) + b_ref[...]
    x_ref[...] = x
    rows = j * RB + lax.broadcasted_iota(jnp.int32, (RB, 1), 0)
    xm = jnp.where(rows < NVAL, x, 0.0)

    @pl.when(j == 0)
    def _():
        st_ref[...] = jnp.zeros_like(st_ref)

    st_ref[0:1, :] += jnp.sum(xm, axis=0, keepdims=True)
    st_ref[1:2, :] += jnp.sum(xm * xm, axis=0, keepdims=True)


def _xlayer(tmpI, aggI, Wexp, b512):
    return pl.pallas_call(
        _xlayer_body,
        grid=(GB,),
        in_specs=([_iqspec(q) for q in range(NQ)]
                  + [_iqspec(q) for q in range(NQ)]
                  + [_cspec((NQ * 128, 512)), _cspec((1, 512))]),
        out_specs=[
            pl.BlockSpec((RB, 512), lambda j: (j, 0)),
            pl.BlockSpec((8, 512), lambda j: (0, 0)),
        ],
        out_shape=[
            jax.ShapeDtypeStruct((NR, 512), jnp.float32),
            jax.ShapeDtypeStruct((8, 512), jnp.float32),
        ],
    )(*([tmpI] * NQ), *([aggI] * NQ), Wexp, b512)


def _bn512(st_ref, F_ref, G_ref, g_ref, b_ref):
    m = jnp.dot(st_ref[0:1, :], F_ref[...],
                preferred_element_type=jnp.float32) * (1.0 / N)
    em2 = jnp.dot(st_ref[1:2, :], F_ref[...],
                  preferred_element_type=jnp.float32) * (1.0 / N)
    inv = lax.rsqrt(em2 - m * m + 1e-5)
    scale = jnp.dot(inv, G_ref[...], preferred_element_type=jnp.float32)
    off = jnp.dot(m * inv, G_ref[...], preferred_element_type=jnp.float32)
    return scale * g_ref[...], b_ref[...] - off * g_ref[...]


def _ylayer_body(*refs):
    st_ref, x_ref, F_ref, G_ref = refs[0:4]
    ts = refs[4:4 + NQ]
    g_ref, b_ref, w_ref, wx_ref, b1_ref, y_ref, sty_ref = refs[4 + NQ:]
    j = pl.program_id(0)
    sc, of = _bn512(st_ref, F_ref, G_ref, g_ref, b_ref)
    xn = _leaky(x_ref[...] * sc + of)
    tcat = jnp.concatenate([ts[q][0] for q in range(NQ)], axis=1)
    y = (jnp.dot(tcat, w_ref[...], preferred_element_type=jnp.float32)
         + jnp.dot(xn, wx_ref[...], preferred_element_type=jnp.float32)
         + b1_ref[...])
    y_ref[...] = y
    rows = j * RB + lax.broadcasted_iota(jnp.int32, (RB, 1), 0)
    ym = jnp.where(rows < NVAL, y, 0.0)

    @pl.when(j == 0)
    def _():
        sty_ref[...] = jnp.zeros_like(sty_ref)

    sty_ref[0:1, :] += jnp.sum(ym, axis=0, keepdims=True)
    sty_ref[1:2, :] += jnp.sum(ym * ym, axis=0, keepdims=True)


def _ylayer(st_x, x2, tmpI, F, G, g512, b512, W1e, W1x, b1_512):
    return pl.pallas_call(
        _ylayer_body,
        grid=(GB,),
        in_specs=([_cspec((8, 512)),
                   pl.BlockSpec((RB, 512), lambda j: (j, 0)),
                   _cspec((512, DIM)), _cspec((DIM, 512))]
                  + [_iqspec(q) for q in range(NQ)]
                  + [_cspec((1, 512)), _cspec((1, 512)),
                     _cspec((NQ * 128, 512)), _cspec((512, 512)),
                     _cspec((1, 512))]),
        out_specs=[
            pl.BlockSpec((RB, 512), lambda j: (j, 0)),
            pl.BlockSpec((8, 512), lambda j: (0, 0)),
        ],
        out_shape=[
            jax.ShapeDtypeStruct((NR, 512), jnp.float32),
            jax.ShapeDtypeStruct((8, 512), jnp.float32),
        ],
    )(st_x, x2, F, G, *([tmpI] * NQ), g512, b512, W1e, W1x, b1_512)


def _zlayer_body(st_ref, y_ref, F_ref, G_ref, g_ref, b_ref, w2_ref, b2_ref,
                 z_ref):
    sc, of = _bn512(st_ref, F_ref, G_ref, g_ref, b_ref)
    yn = _leaky(y_ref[...] * sc + of)
    z = jnp.dot(yn, w2_ref[...], preferred_element_type=jnp.float32) + b2_ref[...]
    z_ref[...] = 1.0 / (1.0 + jnp.exp(-z))


def _zlayer(st_y, y2, F, G, g512, b512, W2e, b2):
    return pl.pallas_call(
        _zlayer_body,
        grid=(GB,),
        in_specs=[_cspec((8, 512)),
                  pl.BlockSpec((RB, 512), lambda j: (j, 0)),
                  _cspec((512, DIM)), _cspec((DIM, 512)),
                  _cspec((1, 512)), _cspec((1, 512)),
                  _cspec((512, IL)), _cspec((1, 1))],
        out_specs=pl.BlockSpec((RB, IL), lambda j: (j, 0)),
        out_shape=jax.ShapeDtypeStruct((NR, IL), jnp.float32),
    )(st_y, y2, F, G, g512, b512, W2e, b2)


def kernel(node_deg, node_lab, edge_index, embed_deg_w, embed_lab_w,
           W0, b0, bn0_g, bn0_b, fc_W1, fc_b1, fc_bn_g, fc_bn_b, fc_W2, fc_b2):
    i32 = jnp.int32
    f32 = jnp.float32
    npad = NPAD - N
    vals2 = jnp.stack([
        jnp.concatenate([node_deg.astype(i32), jnp.zeros((npad,), i32)]),
        jnp.concatenate([node_lab.astype(i32), jnp.zeros((npad,), i32)]),
    ]).reshape(2, NS, RPT)
    tabs = jnp.stack([embed_deg_w,
                      jnp.pad(embed_lab_w, ((0, 64 - embed_lab_w.shape[0]), (0, 0)))])
    tabs8 = tabs.reshape(2, 64, PPC, QD).transpose(0, 2, 1, 3).reshape(NQ, 64, QD)

    epad = EPAD - E
    srcp = jnp.concatenate([edge_index[0].astype(i32), jnp.zeros((epad,), i32)])
    dstp = jnp.concatenate([edge_index[1].astype(i32), jnp.full((epad,), N, i32)])
    src2d = srcp.reshape(NS, EPT)
    dst2d = dstp.reshape(NS, CPT, CHUNK)
    zer = jnp.zeros((RPT, QD), f32)

    tmp8 = _sc_embed(vals2, tabs8)
    agg8 = _sc_segsum(src2d, dst2d, tmp8, zer)
    tmpI = tmp8.reshape(NQ, NR, 128)
    aggI = agg8.reshape(NQ, NR, 128)

    eye16 = jnp.eye(IL, dtype=f32)
    Wexp = jnp.concatenate(
        [jnp.kron(eye16, W0[QD * q:QD * (q + 1), :]) for q in range(NQ)], axis=0)
    W1e = jnp.concatenate(
        [jnp.kron(eye16, fc_W1[QD * q:QD * (q + 1), :]) for q in range(NQ)], axis=0)
    W1x = jnp.kron(eye16, fc_W1[2 * DIM:, :])
    W2e = jnp.kron(eye16, fc_W2)
    F = jnp.tile(jnp.eye(DIM, dtype=f32), (IL, 1))
    G = jnp.tile(jnp.eye(DIM, dtype=f32), (1, IL))

    def t16(v):
        return jnp.tile(v, IL).reshape(1, 512)

    x2, st_x = _xlayer(tmpI, aggI, Wexp, t16(b0))
    y2, st_y = _ylayer(st_x, x2, tmpI, F, G, t16(bn0_g), t16(bn0_b),
                       W1e, W1x, t16(fc_b1))
    z2 = _zlayer(st_y, y2, F, G, t16(fc_bn_g), t16(fc_bn_b), W2e,
                 fc_b2.reshape(1, 1))
    return z2.reshape(NPAD)[:N]
